# speedup vs baseline: 12.4171x; 12.4171x over previous
import jax
import jax.numpy as jnp
from jax.experimental import pallas as pl
from jax.experimental.pallas import tpu as pltpu

G = 64


def _body(x_ref, seg_ref, w_ref, out_ref, m_ref, d_ref, acc_ref):
    i = pl.program_id(0)
    nblk = pl.num_programs(0)

    @pl.when(i == 0)
    def _init():
        m_ref[...] = jnp.full(m_ref.shape, -jnp.inf, jnp.float32)
        d_ref[...] = jnp.zeros(d_ref.shape, jnp.float32)
        acc_ref[...] = jnp.zeros(acc_ref.shape, jnp.float32)

    x_blk = x_ref[...]
    B = x_blk.shape[0]
    l = jnp.dot(x_blk, w_ref[...],
                preferred_element_type=jnp.float32)
    seg = seg_ref[0, 0, :].reshape(B, 1)
    gids = jax.lax.broadcasted_iota(jnp.int32, (B, G), 1)
    onehot_b = seg == gids
    onehot = onehot_b.astype(jnp.float32)

    neg_inf = jnp.float32(-jnp.inf)
    m_blk = jnp.max(jnp.where(onehot_b, l, neg_inf), axis=0, keepdims=True)
    m_old = m_ref[...]
    m_new = jnp.maximum(m_old, m_blk)
    alpha = jnp.where(m_new == neg_inf, 1.0, jnp.exp(m_old - m_new))

    m_row = jnp.max(jnp.where(onehot_b, m_new, neg_inf), axis=1, keepdims=True)
    p = jnp.exp(l - m_row)

    pd = jax.lax.dot_general(p, onehot, (((0,), (0,)), ((), ())),
                             preferred_element_type=jnp.float32)
    px = p * x_blk
    contrib = jax.lax.dot_general(onehot, px, (((0,), (0,)), ((), ())),
                                  preferred_element_type=jnp.float32)

    m_ref[...] = m_new
    d_ref[...] = alpha * d_ref[...] + pd
    acc_ref[...] = jnp.reshape(alpha, (G, 1)) * acc_ref[...] + contrib

    @pl.when(i == nblk - 1)
    def _finish():
        out_ref[...] = acc_ref[...] / (d_ref[...].reshape(G, 1) + 1e-16)


def kernel(x, batch, W, b):
    del b
    N, D = x.shape
    B = 2000
    nblk = N // B
    seg3 = batch.astype(jnp.int32).reshape(nblk, 1, B)

    return pl.pallas_call(
        _body,
        grid=(nblk,),
        in_specs=[
            pl.BlockSpec((B, D), lambda i: (i, 0)),
            pl.BlockSpec((1, 1, B), lambda i: (i, 0, 0)),
            pl.BlockSpec((D, 1), lambda i: (0, 0)),
        ],
        out_specs=pl.BlockSpec((G, D), lambda i: (0, 0)),
        out_shape=jax.ShapeDtypeStruct((G, D), jnp.float32),
        scratch_shapes=[
            pltpu.VMEM((1, G), jnp.float32),
            pltpu.VMEM((1, G), jnp.float32),
            pltpu.VMEM((G, D), jnp.float32),
        ],
    )(x, seg3, W)
